# baseline (device time: 21416 ns/iter reference)
import jax
import jax.numpy as jnp
from jax import lax
from jax.experimental import pallas as pl
from jax.experimental.pallas import tpu as pltpu

N_DEV = 32


def _coords(k: int) -> tuple[int, int, int]:
    z, p = divmod(k, 8)
    y, xi = divmod(p, 2)
    x = xi if y % 2 == 0 else 1 - xi
    return x, y, z


def _avg_dist(offset: int) -> float:
    tot = 0
    for a in range(N_DEV):
        xa, ya, za = _coords(a)
        xb, yb, zb = _coords((a + offset) % N_DEV)
        tot += abs(xa - xb) + abs(ya - yb) + abs(za - zb)
    return tot / N_DEV


_NEAR_FIRST = sorted(range(1, N_DEV), key=_avg_dist)
_FAR_FIRST = list(reversed(_NEAR_FIRST))


def kernel(x):
    m, n = x.shape
    rows = m // N_DEV

    def body(x_ref, out_ref, stage, rs_buf, red_buf,
             ready_sems, sems1_s, sems1_r, sems2_s, sems2_r):
        my_pos = lax.axis_index("i")

        barrier_sem = pltpu.get_barrier_semaphore()
        pl.semaphore_signal(barrier_sem, inc=1)
        pl.semaphore_wait(barrier_sem, 1)

        for j in _FAR_FIRST:
            pl.semaphore_signal(
                ready_sems.at[N_DEV - j],
                inc=1,
                device_id=((my_pos + j) % N_DEV,),
                device_id_type=pl.DeviceIdType.MESH,
            )

        stage[:, :] = x_ref[:, :].astype(stage.dtype)
        rs_buf[0, :, :] = stage[pl.ds(my_pos * rows, rows), :]

        p1 = []
        for k in _NEAR_FIRST:
            dst = (my_pos + k) % N_DEV
            pl.semaphore_wait(ready_sems.at[k], 1)
            rdma = pltpu.make_async_remote_copy(
                src_ref=stage.at[pl.ds(dst * rows, rows), :],
                dst_ref=rs_buf.at[k],
                send_sem=sems1_s.at[k],
                recv_sem=sems1_r.at[k],
                device_id=(dst,),
                device_id_type=pl.DeviceIdType.MESH,
            )
            rdma.start()
            p1.append(rdma)

        for rdma in p1:
            rdma.wait_recv()

        red = jnp.sum(rs_buf[:, :, :].astype(jnp.float32), axis=0)
        red_bf = red.astype(red_buf.dtype)
        red_buf[:, :] = red_bf
        out_ref[pl.ds(my_pos * rows, rows), :] = red_bf

        p2 = []
        for k in _FAR_FIRST:
            dst = (my_pos + k) % N_DEV
            rdma = pltpu.make_async_remote_copy(
                src_ref=red_buf,
                dst_ref=out_ref.at[pl.ds(my_pos * rows, rows), :],
                send_sem=sems2_s.at[k],
                recv_sem=sems2_r.at[k],
                device_id=(dst,),
                device_id_type=pl.DeviceIdType.MESH,
            )
            rdma.start()
            p2.append(rdma)

        for k in _NEAR_FIRST:
            src = (my_pos - k) % N_DEV
            recv = pltpu.make_async_remote_copy(
                src_ref=red_buf,
                dst_ref=out_ref.at[pl.ds(src * rows, rows), :],
                send_sem=sems2_s.at[k],
                recv_sem=sems2_r.at[k],
                device_id=(src,),
                device_id_type=pl.DeviceIdType.MESH,
            )
            recv.wait_recv()

        for rdma in p1:
            rdma.wait_send()
        for rdma in p2:
            rdma.wait_send()

    return pl.pallas_call(
        body,
        out_shape=jax.ShapeDtypeStruct((m, n), jnp.bfloat16),
        in_specs=[pl.BlockSpec(memory_space=pltpu.VMEM)],
        out_specs=pl.BlockSpec(memory_space=pltpu.VMEM),
        scratch_shapes=[
            pltpu.VMEM((m, n), jnp.bfloat16),
            pltpu.VMEM((N_DEV, m // N_DEV, n), jnp.bfloat16),
            pltpu.VMEM((m // N_DEV, n), jnp.bfloat16),
            pltpu.SemaphoreType.REGULAR((N_DEV,)),
            pltpu.SemaphoreType.DMA((N_DEV,)),
            pltpu.SemaphoreType.DMA((N_DEV,)),
            pltpu.SemaphoreType.DMA((N_DEV,)),
            pltpu.SemaphoreType.DMA((N_DEV,)),
        ],
        compiler_params=pltpu.CompilerParams(collective_id=0),
    )(x)


# device time: 20577 ns/iter; 1.0408x vs baseline; 1.0408x over previous
import jax
import jax.numpy as jnp
from jax import lax
from jax.experimental import pallas as pl
from jax.experimental.pallas import tpu as pltpu

N_DEV = 32


_NEAR_FIRST = list(range(1, N_DEV))
_FAR_FIRST = list(range(1, N_DEV))


def kernel(x):
    m, n = x.shape
    rows = m // N_DEV

    def body(x_ref, out_ref, stage, rs_buf, red_buf,
             ready_sems, sems1_s, sems1_r, sems2_s, sems2_r):
        my_pos = lax.axis_index("i")

        barrier_sem = pltpu.get_barrier_semaphore()
        pl.semaphore_signal(barrier_sem, inc=1)
        pl.semaphore_wait(barrier_sem, 1)

        for j in _FAR_FIRST:
            pl.semaphore_signal(
                ready_sems.at[N_DEV - j],
                inc=1,
                device_id=((my_pos + j) % N_DEV,),
                device_id_type=pl.DeviceIdType.MESH,
            )

        stage[:, :] = x_ref[:, :].astype(stage.dtype)
        rs_buf[0, :, :] = stage[pl.ds(my_pos * rows, rows), :]

        p1 = []
        for k in _NEAR_FIRST:
            dst = (my_pos + k) % N_DEV
            pl.semaphore_wait(ready_sems.at[k], 1)
            rdma = pltpu.make_async_remote_copy(
                src_ref=stage.at[pl.ds(dst * rows, rows), :],
                dst_ref=rs_buf.at[k],
                send_sem=sems1_s.at[k],
                recv_sem=sems1_r.at[k],
                device_id=(dst,),
                device_id_type=pl.DeviceIdType.MESH,
            )
            rdma.start()
            p1.append(rdma)

        for rdma in p1:
            rdma.wait_recv()

        red = jnp.sum(rs_buf[:, :, :].astype(jnp.float32), axis=0)
        red_bf = red.astype(red_buf.dtype)
        red_buf[:, :] = red_bf
        out_ref[pl.ds(my_pos * rows, rows), :] = red_bf

        p2 = []
        for k in _FAR_FIRST:
            dst = (my_pos + k) % N_DEV
            rdma = pltpu.make_async_remote_copy(
                src_ref=red_buf,
                dst_ref=out_ref.at[pl.ds(my_pos * rows, rows), :],
                send_sem=sems2_s.at[k],
                recv_sem=sems2_r.at[k],
                device_id=(dst,),
                device_id_type=pl.DeviceIdType.MESH,
            )
            rdma.start()
            p2.append(rdma)

        for k in _NEAR_FIRST:
            src = (my_pos - k) % N_DEV
            recv = pltpu.make_async_remote_copy(
                src_ref=red_buf,
                dst_ref=out_ref.at[pl.ds(src * rows, rows), :],
                send_sem=sems2_s.at[k],
                recv_sem=sems2_r.at[k],
                device_id=(src,),
                device_id_type=pl.DeviceIdType.MESH,
            )
            recv.wait_recv()

        for rdma in p1:
            rdma.wait_send()
        for rdma in p2:
            rdma.wait_send()

    return pl.pallas_call(
        body,
        out_shape=jax.ShapeDtypeStruct((m, n), jnp.bfloat16),
        in_specs=[pl.BlockSpec(memory_space=pltpu.VMEM)],
        out_specs=pl.BlockSpec(memory_space=pltpu.VMEM),
        scratch_shapes=[
            pltpu.VMEM((m, n), jnp.bfloat16),
            pltpu.VMEM((N_DEV, m // N_DEV, n), jnp.bfloat16),
            pltpu.VMEM((m // N_DEV, n), jnp.bfloat16),
            pltpu.SemaphoreType.REGULAR((N_DEV,)),
            pltpu.SemaphoreType.DMA((N_DEV,)),
            pltpu.SemaphoreType.DMA((N_DEV,)),
            pltpu.SemaphoreType.DMA((N_DEV,)),
            pltpu.SemaphoreType.DMA((N_DEV,)),
        ],
        compiler_params=pltpu.CompilerParams(collective_id=0),
    )(x)
